# baseline (speedup 1.0000x reference)
import functools

import jax
import jax.numpy as jnp
from jax import lax
from jax.experimental import pallas as pl
from jax.experimental.pallas import tpu as pltpu
from jax.experimental.pallas import tpu_sc as plsc

NUM_FIELDS = 26
VOCAB = 100000
EMBED_DIM = 32
BATCH = 16384

_info = plsc.get_sparse_core_info()
_NC, _NS, _L = _info.num_cores, _info.num_subcores, _info.num_lanes
NW = _NC * _NS
NPAIRS = NUM_FIELDS * EMBED_DIM
PPW = NPAIRS // NW
SLAB = 2048
NSLAB = BATCH // SLAB


def _sc_call(tabT, xP):
    mesh = plsc.VectorSubcoreMesh(core_axis_name="c", subcore_axis_name="s")

    @functools.partial(
        pl.kernel,
        mesh=mesh,
        out_type=jax.ShapeDtypeStruct((NUM_FIELDS, EMBED_DIM, BATCH),
                                      jnp.float32),
        compiler_params=pltpu.CompilerParams(
            needs_layout_passes=False, use_tc_tiling_on_sc=True),
        scratch_types=[
            pltpu.VMEM((VOCAB,), jnp.float32),
            pltpu.VMEM((BATCH // 128, 128), jnp.int32),
            pltpu.VMEM((SLAB,), jnp.float32),
            pltpu.VMEM((SLAB,), jnp.float32),
            pltpu.SemaphoreType.DMA,
            pltpu.SemaphoreType.DMA,
            pltpu.SemaphoreType.DMA,
        ],
    )
    def k(tabT_hbm, xP_hbm, out_hbm, row_v, idx_v, o_v0, o_v1, sem,
          osem0, osem1):
        wid = lax.axis_index("s") * _NC + lax.axis_index("c")
        obufs = (o_v0, o_v1)
        osems = (osem0, osem1)

        def pair_body(p0, prev_f):
            p = wid * PPW + p0
            f = p // EMBED_DIM
            d = p % EMBED_DIM

            @pl.when(f != prev_f)
            def _():
                pltpu.async_copy(xP_hbm.at[f], idx_v, sem).wait()

            pltpu.async_copy(tabT_hbm.at[f, d], row_v, sem).wait()

            for s in range(NSLAB):
                ob = obufs[s % 2]
                if s >= 2:
                    pltpu.make_async_copy(
                        ob, out_hbm.at[f, d, pl.ds((s - 2) * SLAB, SLAB)],
                        osems[s % 2]).wait()
                else:
                    @pl.when(p0 > 0)
                    def _():
                        pltpu.make_async_copy(
                            ob,
                            out_hbm.at[f, d,
                                       pl.ds((NSLAB - 2 + s) * SLAB, SLAB)],
                            osems[s % 2]).wait()

                for j in range(SLAB // 128):
                    base_row = s * (SLAB // 128) + j
                    for u in range(8):
                        idxv = idx_v[base_row, pl.ds(u * _L, _L)]
                        ob[pl.ds((j * 128 + u * _L), _L)] = (
                            plsc.load_gather(row_v, [idxv]))

                pltpu.async_copy(
                    ob, out_hbm.at[f, d, pl.ds(s * SLAB, SLAB)],
                    osems[s % 2])
            return f

        lax.fori_loop(0, PPW, pair_body, jnp.int32(-1))
        for s in (NSLAB - 2, NSLAB - 1):
            pltpu.make_async_copy(
                obufs[s % 2],
                out_hbm.at[NUM_FIELDS - 1, EMBED_DIM - 1,
                           pl.ds(s * SLAB, SLAB)],
                osems[s % 2]).wait()

    return k(tabT, xP)


def kernel(x, tables):
    tabT = jnp.transpose(tables, (0, 2, 1))
    xP = x.T.astype(jnp.int32).reshape(NUM_FIELDS, BATCH // 128, 128)
    outT = _sc_call(tabT, xP)
    return jnp.transpose(outT, (2, 1, 0))


# speedup vs baseline: 1.1209x; 1.1209x over previous
import functools

import jax
import jax.numpy as jnp
from jax import lax
from jax.experimental import pallas as pl
from jax.experimental.pallas import tpu as pltpu
from jax.experimental.pallas import tpu_sc as plsc

NUM_FIELDS = 26
VOCAB = 100000
EMBED_DIM = 32
BATCH = 16384

_info = plsc.get_sparse_core_info()
_NC, _NS, _L = _info.num_cores, _info.num_subcores, _info.num_lanes
NW = _NC * _NS
NPAIRS = NUM_FIELDS * EMBED_DIM
PPW = NPAIRS // NW
SLAB = 2048
NSLAB = BATCH // SLAB


def _sc_call(tabT, xP):
    mesh = plsc.VectorSubcoreMesh(core_axis_name="c", subcore_axis_name="s")

    @functools.partial(
        pl.kernel,
        mesh=mesh,
        out_type=jax.ShapeDtypeStruct((NUM_FIELDS, EMBED_DIM, BATCH),
                                      jnp.float32),
        compiler_params=pltpu.CompilerParams(
            needs_layout_passes=False, use_tc_tiling_on_sc=True),
        scratch_types=[
            pltpu.VMEM((VOCAB,), jnp.float32),
            pltpu.VMEM((BATCH // 128, 128), jnp.int32),
            pltpu.VMEM((SLAB,), jnp.float32),
            pltpu.VMEM((SLAB,), jnp.float32),
            pltpu.SemaphoreType.DMA,
            pltpu.SemaphoreType.DMA,
            pltpu.SemaphoreType.DMA,
        ],
    )
    def k(tabT_hbm, xP_hbm, out_hbm, row_v, idx_v, o_v0, o_v1, sem,
          osem0, osem1):
        wid = lax.axis_index("s") * _NC + lax.axis_index("c")
        obufs = (o_v0, o_v1)
        osems = (osem0, osem1)

        def pair_body(p0, prev_f):
            p = wid * PPW + p0
            f = p // EMBED_DIM
            d = p % EMBED_DIM

            @pl.when(f != prev_f)
            def _():
                pltpu.async_copy(xP_hbm.at[f], idx_v, sem).wait()

            pltpu.async_copy(tabT_hbm.at[f, d], row_v, sem).wait()

            for s in range(NSLAB):
                ob = obufs[s % 2]
                if s >= 2:
                    pltpu.make_async_copy(
                        ob, out_hbm.at[f, d, pl.ds((s - 2) * SLAB, SLAB)],
                        osems[s % 2]).wait()
                else:
                    @pl.when(p0 > 0)
                    def _():
                        pltpu.make_async_copy(
                            ob,
                            out_hbm.at[f, d,
                                       pl.ds((NSLAB - 2 + s) * SLAB, SLAB)],
                            osems[s % 2]).wait()

                def gather_body(j, c2):
                    base_row = s * (SLAB // 128) + j
                    for u in range(8):
                        idxv = idx_v[base_row, pl.ds(u * _L, _L)]
                        ob[pl.ds(j * 128 + u * _L, _L)] = (
                            plsc.load_gather(row_v, [idxv]))
                    return c2

                lax.fori_loop(0, SLAB // 128, gather_body, jnp.int32(0))
                pltpu.async_copy(
                    ob, out_hbm.at[f, d, pl.ds(s * SLAB, SLAB)],
                    osems[s % 2])
            return f

        lax.fori_loop(0, PPW, pair_body, jnp.int32(-1))
        for s in (NSLAB - 2, NSLAB - 1):
            pltpu.make_async_copy(
                obufs[s % 2],
                out_hbm.at[NUM_FIELDS - 1, EMBED_DIM - 1,
                           pl.ds(s * SLAB, SLAB)],
                osems[s % 2]).wait()

    return k(tabT, xP)


def kernel(x, tables):
    tabT = jnp.transpose(tables, (0, 2, 1))
    xP = x.T.astype(jnp.int32).reshape(NUM_FIELDS, BATCH // 128, 128)
    outT = _sc_call(tabT, xP)
    return jnp.transpose(outT, (2, 1, 0))


# speedup vs baseline: 1.1213x; 1.0003x over previous
import functools

import jax
import jax.numpy as jnp
from jax import lax
from jax.experimental import pallas as pl
from jax.experimental.pallas import tpu as pltpu
from jax.experimental.pallas import tpu_sc as plsc

NUM_FIELDS = 26
VOCAB = 100000
EMBED_DIM = 32
BATCH = 16384

_info = plsc.get_sparse_core_info()
_NC, _NS, _L = _info.num_cores, _info.num_subcores, _info.num_lanes
NW = _NC * _NS
NPAIRS = NUM_FIELDS * EMBED_DIM
PPW = NPAIRS // NW
SLAB = 2048
NSLAB = BATCH // SLAB


def _sc_call(tabT, xP):
    mesh = plsc.VectorSubcoreMesh(core_axis_name="c", subcore_axis_name="s")

    @functools.partial(
        pl.kernel,
        mesh=mesh,
        out_type=jax.ShapeDtypeStruct((NUM_FIELDS, EMBED_DIM, BATCH),
                                      jnp.float32),
        compiler_params=pltpu.CompilerParams(
            needs_layout_passes=False, use_tc_tiling_on_sc=True),
        scratch_types=[
            pltpu.VMEM((VOCAB,), jnp.float32),
            pltpu.VMEM((BATCH // 128, 128), jnp.int32),
            pltpu.VMEM((SLAB,), jnp.float32),
            pltpu.VMEM((SLAB,), jnp.float32),
            pltpu.SemaphoreType.DMA,
            pltpu.SemaphoreType.DMA,
            pltpu.SemaphoreType.DMA,
        ],
    )
    def k(tabT_hbm, xP_hbm, out_hbm, row_v, idx_v, o_v0, o_v1, sem,
          osem0, osem1):
        wid = lax.axis_index("s") * _NC + lax.axis_index("c")
        obufs = (o_v0, o_v1)
        osems = (osem0, osem1)

        def pair_body(p0, prev_f):
            p0r = lax.rem(p0 + wid, PPW)
            p = wid * PPW + p0r
            f = p // EMBED_DIM
            d = p % EMBED_DIM

            @pl.when(f != prev_f)
            def _():
                pltpu.async_copy(xP_hbm.at[f], idx_v, sem).wait()

            pltpu.async_copy(tabT_hbm.at[f, d], row_v, sem).wait()

            for s in range(NSLAB):
                ob = obufs[s % 2]
                if s >= 2:
                    pltpu.make_async_copy(
                        ob, out_hbm.at[f, d, pl.ds((s - 2) * SLAB, SLAB)],
                        osems[s % 2]).wait()
                else:
                    @pl.when(p0 > 0)
                    def _():
                        pltpu.make_async_copy(
                            ob,
                            out_hbm.at[f, d,
                                       pl.ds((NSLAB - 2 + s) * SLAB, SLAB)],
                            osems[s % 2]).wait()

                def gather_body(j, c2):
                    base_row = s * (SLAB // 128) + j
                    for u in range(8):
                        idxv = idx_v[base_row, pl.ds(u * _L, _L)]
                        ob[pl.ds(j * 128 + u * _L, _L)] = (
                            plsc.load_gather(row_v, [idxv]))
                    return c2

                lax.fori_loop(0, SLAB // 128, gather_body, jnp.int32(0))
                pltpu.async_copy(
                    ob, out_hbm.at[f, d, pl.ds(s * SLAB, SLAB)],
                    osems[s % 2])
            return f

        lax.fori_loop(0, PPW, pair_body, jnp.int32(-1))
        for s in (NSLAB - 2, NSLAB - 1):
            pltpu.make_async_copy(
                obufs[s % 2],
                out_hbm.at[NUM_FIELDS - 1, EMBED_DIM - 1,
                           pl.ds(s * SLAB, SLAB)],
                osems[s % 2]).wait()

    return k(tabT, xP)


def kernel(x, tables):
    tabT = jnp.transpose(tables, (0, 2, 1))
    xP = x.T.astype(jnp.int32).reshape(NUM_FIELDS, BATCH // 128, 128)
    outT = _sc_call(tabT, xP)
    return jnp.transpose(outT, (2, 1, 0))


# speedup vs baseline: 1.1287x; 1.0066x over previous
import functools

import jax
import jax.numpy as jnp
from jax import lax
from jax.experimental import pallas as pl
from jax.experimental.pallas import tpu as pltpu
from jax.experimental.pallas import tpu_sc as plsc

NUM_FIELDS = 26
VOCAB = 100000
EMBED_DIM = 32
BATCH = 16384

_info = plsc.get_sparse_core_info()
_NC, _NS, _L = _info.num_cores, _info.num_subcores, _info.num_lanes
NW = _NC * _NS
NPAIRS = NUM_FIELDS * EMBED_DIM
PPW = NPAIRS // NW
SLAB = 4096
NSLAB = BATCH // SLAB


def _sc_call(tabT, xP):
    mesh = plsc.VectorSubcoreMesh(core_axis_name="c", subcore_axis_name="s")

    @functools.partial(
        pl.kernel,
        mesh=mesh,
        out_type=jax.ShapeDtypeStruct((NUM_FIELDS, EMBED_DIM, BATCH),
                                      jnp.float32),
        compiler_params=pltpu.CompilerParams(
            needs_layout_passes=False, use_tc_tiling_on_sc=True),
        scratch_types=[
            pltpu.VMEM((VOCAB,), jnp.float32),
            pltpu.VMEM((BATCH // 128, 128), jnp.int32),
            pltpu.VMEM((SLAB,), jnp.float32),
            pltpu.VMEM((SLAB,), jnp.float32),
            pltpu.SemaphoreType.DMA,
            pltpu.SemaphoreType.DMA,
            pltpu.SemaphoreType.DMA,
        ],
    )
    def k(tabT_hbm, xP_hbm, out_hbm, row_v, idx_v, o_v0, o_v1, sem,
          osem0, osem1):
        wid = lax.axis_index("s") * _NC + lax.axis_index("c")
        obufs = (o_v0, o_v1)
        osems = (osem0, osem1)

        def pair_body(p0, prev_f):
            p0r = lax.rem(p0 + wid, PPW)
            p = wid * PPW + p0r
            f = p // EMBED_DIM
            d = p % EMBED_DIM

            @pl.when(f != prev_f)
            def _():
                pltpu.async_copy(xP_hbm.at[f], idx_v, sem).wait()

            pltpu.async_copy(tabT_hbm.at[f, d], row_v, sem).wait()

            for s in range(NSLAB):
                ob = obufs[s % 2]
                if s >= 2:
                    pltpu.make_async_copy(
                        ob, out_hbm.at[f, d, pl.ds((s - 2) * SLAB, SLAB)],
                        osems[s % 2]).wait()
                else:
                    @pl.when(p0 > 0)
                    def _():
                        pltpu.make_async_copy(
                            ob,
                            out_hbm.at[f, d,
                                       pl.ds((NSLAB - 2 + s) * SLAB, SLAB)],
                            osems[s % 2]).wait()

                def gather_body(j, c2):
                    base_row = s * (SLAB // 128) + j
                    for u in range(8):
                        idxv = idx_v[base_row, pl.ds(u * _L, _L)]
                        ob[pl.ds(j * 128 + u * _L, _L)] = (
                            plsc.load_gather(row_v, [idxv]))
                    return c2

                lax.fori_loop(0, SLAB // 128, gather_body, jnp.int32(0))
                pltpu.async_copy(
                    ob, out_hbm.at[f, d, pl.ds(s * SLAB, SLAB)],
                    osems[s % 2])
            return f

        lax.fori_loop(0, PPW, pair_body, jnp.int32(-1))
        for s in (NSLAB - 2, NSLAB - 1):
            pltpu.make_async_copy(
                obufs[s % 2],
                out_hbm.at[NUM_FIELDS - 1, EMBED_DIM - 1,
                           pl.ds(s * SLAB, SLAB)],
                osems[s % 2]).wait()

    return k(tabT, xP)


def kernel(x, tables):
    tabT = jnp.transpose(tables, (0, 2, 1))
    xP = x.T.astype(jnp.int32).reshape(NUM_FIELDS, BATCH // 128, 128)
    outT = _sc_call(tabT, xP)
    return jnp.transpose(outT, (2, 1, 0))
